# baseline (speedup 1.0000x reference)
import jax
import jax.numpy as jnp
from jax.experimental import pallas as pl
from jax.experimental.pallas import tpu as pltpu

_K = 8192
_D = 32
_BETA = 0.25
_MB = 512
_N_ROWS = 16384
_GRID = _N_ROWS // _MB


def _vq_block(idx_ref, z_ref, e_ref, zq_ref, acc_ref):
    i = pl.program_id(0)
    zb = z_ref[...]
    e = e_ref[...]
    idx = idx_ref[0, 0, :]
    col = jax.lax.broadcasted_iota(jnp.int32, (_MB, _K), 1)
    enc = (col == idx[:, None]).astype(jnp.float32)
    zq = jax.lax.dot_general(
        enc, e, (((1,), (1,)), ((), ())),
        preferred_element_type=jnp.float32)
    zq_ref[...] = zb + (zq - zb)
    s = jnp.sum((zq - zb) ** 2)

    @pl.when(i == 0)
    def _init():
        acc_ref[...] = jnp.zeros_like(acc_ref)

    acc_ref[...] += s


def kernel(z, embedding):
    z_flat = jnp.reshape(z, (-1, _D))
    a = jnp.sum(z_flat ** 2, axis=1, keepdims=True)
    c = jnp.sum(embedding ** 2, axis=0)
    m = jax.lax.dot_general(
        z_flat.astype(jnp.bfloat16), embedding, (((1,), (0,)), ((), ())),
        preferred_element_type=jnp.float32)
    d = a + c - 2.0 * m
    min_encoding_indices = jnp.argmin(d, axis=1)

    idx3 = jnp.reshape(min_encoding_indices, (_GRID, 1, _MB))
    zq, acc = pl.pallas_call(
        _vq_block,
        grid=(_GRID,),
        in_specs=[
            pl.BlockSpec((1, 1, _MB), lambda i: (i, 0, 0)),
            pl.BlockSpec((_MB, _D), lambda i: (i, 0)),
            pl.BlockSpec((_D, _K), lambda i: (0, 0)),
        ],
        out_specs=[
            pl.BlockSpec((_MB, _D), lambda i: (i, 0)),
            pl.BlockSpec((8, 128), lambda i: (0, 0)),
        ],
        out_shape=[
            jax.ShapeDtypeStruct((_N_ROWS, _D), jnp.float32),
            jax.ShapeDtypeStruct((8, 128), jnp.float32),
        ],
    )(idx3, z_flat, embedding)

    z_q = jnp.reshape(zq, z.shape)
    mean_sq = acc[0, 0] / jnp.float32(_N_ROWS * _D)
    loss = _BETA * mean_sq + mean_sq
    return (z_q, min_encoding_indices, loss)


# speedup vs baseline: 1.1349x; 1.1349x over previous
import functools
import jax
import jax.numpy as jnp
from jax import lax
from jax.experimental import pallas as pl
from jax.experimental.pallas import tpu as pltpu
from jax.experimental.pallas import tpu_sc as plsc

_K = 8192
_D = 32
_BETA = 0.25
_N_ROWS = 16384
_NC = 2
_NS = 16
_NW = _NC * _NS
_BPW = _N_ROWS // _NW
_CH = 128
_NCH = _BPW // _CH
_L = 16


def _make_sc_kernel():
    mesh = plsc.VectorSubcoreMesh(core_axis_name="c", subcore_axis_name="s")

    @functools.partial(
        pl.kernel, mesh=mesh,
        compiler_params=pltpu.CompilerParams(use_tc_tiling_on_sc=False),
        out_type=[
            jax.ShapeDtypeStruct((_N_ROWS, _D), jnp.float32),
            jax.ShapeDtypeStruct((_NW, _L), jnp.float32),
        ],
        scratch_types=[
            pltpu.VMEM((_NCH, _CH), jnp.int32),
            pltpu.VMEM((_BPW, _D), jnp.float32),
            pltpu.VMEM((_BPW, _D), jnp.float32),
            pltpu.VMEM((_L,), jnp.float32),
            pltpu.SemaphoreType.DMA,
        ],
    )
    def sc_kernel(table_hbm, idx_hbm, z_hbm, zq_hbm, lp_hbm,
                  idx_v, rows_v, z_v, acc_v, sem):
        wid = lax.axis_index("s") * _NC + lax.axis_index("c")
        base = wid * _BPW
        pltpu.sync_copy(idx_hbm.at[pl.ds(wid * _NCH, _NCH)], idx_v)
        copies = []
        for k in range(_NCH):
            copies.append(pltpu.async_copy(
                table_hbm.at[idx_v.at[k]],
                rows_v.at[pl.ds(k * _CH, _CH)], sem))
        pltpu.sync_copy(z_hbm.at[pl.ds(base, _BPW)], z_v)
        for cp in copies:
            cp.wait()

        def body(r, acc):
            for h in range(_D // _L):
                q = rows_v[r, pl.ds(h * _L, _L)]
                zz = z_v[r, pl.ds(h * _L, _L)]
                rows_v[r, pl.ds(h * _L, _L)] = zz + (q - zz)
                acc = acc + (q - zz) * (q - zz)
            return acc

        acc = lax.fori_loop(0, _BPW, body, jnp.zeros((_L,), jnp.float32))
        acc_v[...] = acc
        pltpu.sync_copy(rows_v, zq_hbm.at[pl.ds(base, _BPW)])
        pltpu.sync_copy(acc_v, lp_hbm.at[wid])

    return sc_kernel


_SC_KERNEL = _make_sc_kernel()


def kernel(z, embedding):
    z_flat = jnp.reshape(z, (-1, _D))
    a = jnp.sum(z_flat ** 2, axis=1, keepdims=True)
    c = jnp.sum(embedding ** 2, axis=0)
    m = jax.lax.dot_general(
        z_flat.astype(jnp.bfloat16), embedding, (((1,), (0,)), ((), ())),
        preferred_element_type=jnp.float32)
    d = a + c - 2.0 * m
    min_encoding_indices = jnp.argmin(d, axis=1)

    table = embedding.T
    idx2d = jnp.reshape(min_encoding_indices, (_NW * _NCH, _CH))
    zq, lp = _SC_KERNEL(table, idx2d, z_flat)
    z_q = jnp.reshape(zq, z.shape)
    mean_sq = jnp.sum(lp) / jnp.float32(_N_ROWS * _D)
    loss = _BETA * mean_sq + mean_sq
    return (z_q, min_encoding_indices, loss)
